# Initial kernel scaffold; baseline (speedup 1.0000x reference)
import jax
import jax.numpy as jnp
from jax.experimental import pallas as pl


def kernel(z, edge_index, edge_type, rel_emb):
    raise NotImplementedError("write your pallas kernel here")


# speedup vs baseline: 1.0061x; 1.0061x over previous
import functools

import jax
import jax.numpy as jnp
from jax import lax
from jax.experimental import pallas as pl
from jax.experimental.pallas import tpu as pltpu
from jax.experimental.pallas import tpu_sc as plsc

E = 320000
H = 128
NW = 32
EPW = E // NW
C = 80
NCH = EPW // C
G = C // 16

_mesh = plsc.VectorSubcoreMesh(core_axis_name="c", subcore_axis_name="s")


@functools.partial(
    pl.kernel,
    out_type=jax.ShapeDtypeStruct((E,), jnp.float32),
    mesh=_mesh,
    compiler_params=pltpu.CompilerParams(needs_layout_passes=False),
    scratch_types=[
        pltpu.VMEM((C,), jnp.int32),
        pltpu.VMEM((C,), jnp.int32),
        pltpu.VMEM((C,), jnp.int32),
        pltpu.VMEM((C, H), jnp.float32),
        pltpu.VMEM((C, H), jnp.float32),
        pltpu.VMEM((C, H), jnp.float32),
        pltpu.VMEM((C,), jnp.float32),
        pltpu.SemaphoreType.DMA,
    ],
)
def _transe(z_h, src_h, dst_h, et_h, rel_h, out_h, si, di, ti, ba, bb, br, bo, sem):
    wid = lax.axis_index("s") * 2 + lax.axis_index("c")
    base = wid * EPW

    def chunk(i, carry):
        off = base + i * C
        pltpu.sync_copy(src_h.at[pl.ds(off, C)], si)
        pltpu.sync_copy(dst_h.at[pl.ds(off, C)], di)
        pltpu.sync_copy(et_h.at[pl.ds(off, C)], ti)
        pltpu.async_copy(z_h.at[si], ba, sem).wait()
        pltpu.async_copy(z_h.at[di], bb, sem).wait()
        pltpu.async_copy(rel_h.at[ti], br, sem).wait()

        def group(g, carry2):
            rows = g * 16 + lax.iota(jnp.int32, 16)
            acc = jnp.zeros((16,), jnp.float32)
            for f in range(H):
                fv = jnp.full((16,), f, jnp.int32)
                va = plsc.load_gather(ba, [rows, fv])
                vb = plsc.load_gather(bb, [rows, fv])
                vr = plsc.load_gather(br, [rows, fv])
                d = va + vr - vb
                acc = acc + d * d
            ibits = plsc.bitcast(acc, jnp.int32)
            magic = jnp.full((16,), 0x5F3759DF, jnp.int32)
            y = plsc.bitcast(magic - (ibits >> 1), jnp.float32)
            for _ in range(3):
                y = y * (1.5 - 0.5 * acc * y * y)
            res = jnp.where(acc > 0.0, -(acc * y), 0.0)
            bo[pl.ds(g * 16, 16)] = res
            return carry2

        lax.fori_loop(0, G, group, 0)
        pltpu.sync_copy(bo, out_h.at[pl.ds(off, C)])
        return carry

    lax.fori_loop(0, NCH, chunk, 0)


def kernel(z, edge_index, edge_type, rel_emb):
    src = edge_index[0].astype(jnp.int32)
    dst = edge_index[1].astype(jnp.int32)
    et = edge_type.astype(jnp.int32)
    return _transe(z, src, dst, et, rel_emb)


# speedup vs baseline: 1.2013x; 1.1940x over previous
import functools

import jax
import jax.numpy as jnp
from jax import lax
from jax.experimental import pallas as pl
from jax.experimental.pallas import tpu as pltpu
from jax.experimental.pallas import tpu_sc as plsc

E = 320000
H = 128
NW = 32
EPW = E // NW
C = 80
NCH = EPW // C
NPAIR = (NCH - 1) // 2
G = C // 16

_mesh = plsc.VectorSubcoreMesh(core_axis_name="c", subcore_axis_name="s")

_slot_types = [
    pltpu.VMEM((C,), jnp.int32),
    pltpu.VMEM((C,), jnp.int32),
    pltpu.VMEM((C,), jnp.int32),
    pltpu.VMEM((C, H), jnp.float32),
    pltpu.VMEM((C, H), jnp.float32),
    pltpu.VMEM((C, H), jnp.float32),
    pltpu.VMEM((C,), jnp.float32),
    pltpu.SemaphoreType.DMA,
]


@functools.partial(
    pl.kernel,
    out_type=jax.ShapeDtypeStruct((E,), jnp.float32),
    mesh=_mesh,
    compiler_params=pltpu.CompilerParams(needs_layout_passes=False),
    scratch_types=_slot_types + _slot_types,
)
def _transe(z_h, src_h, dst_h, et_h, rel_h, out_h, *scratch):
    slots = (scratch[:8], scratch[8:])
    wid = lax.axis_index("s") * 2 + lax.axis_index("c")
    base = wid * EPW

    def fire(ci, s):
        si, di, ti, a, b, r, _, sem = s
        off = base + ci * C
        pltpu.sync_copy(src_h.at[pl.ds(off, C)], si)
        pltpu.sync_copy(dst_h.at[pl.ds(off, C)], di)
        pltpu.sync_copy(et_h.at[pl.ds(off, C)], ti)
        pltpu.make_async_copy(z_h.at[si], a, sem).start()
        pltpu.make_async_copy(z_h.at[di], b, sem).start()
        pltpu.make_async_copy(rel_h.at[ti], r, sem).start()

    def finish(ci, s):
        si, di, ti, a, b, r, o, sem = s
        pltpu.make_async_copy(z_h.at[si], a, sem).wait()
        pltpu.make_async_copy(z_h.at[di], b, sem).wait()
        pltpu.make_async_copy(rel_h.at[ti], r, sem).wait()

        def group(g, carry):
            rows = g * 16 + lax.iota(jnp.int32, 16)
            acc = jnp.zeros((16,), jnp.float32)
            for f in range(H):
                fv = jnp.full((16,), f, jnp.int32)
                va = plsc.load_gather(a, [rows, fv])
                vb = plsc.load_gather(b, [rows, fv])
                vr = plsc.load_gather(r, [rows, fv])
                d = va + vr - vb
                acc = acc + d * d
            ibits = plsc.bitcast(acc, jnp.int32)
            magic = jnp.full((16,), 0x5F3759DF, jnp.int32)
            y = plsc.bitcast(magic - (ibits >> 1), jnp.float32)
            for _ in range(3):
                y = y * (1.5 - 0.5 * acc * y * y)
            res = jnp.where(acc > 0.0, -(acc * y), 0.0)
            o[pl.ds(g * 16, 16)] = res
            return carry

        lax.fori_loop(0, G, group, 0)
        pltpu.sync_copy(o, out_h.at[pl.ds(base + ci * C, C)])

    fire(0, slots[0])

    def pair(j, carry):
        c0 = j * 2
        fire(c0 + 1, slots[1])
        finish(c0, slots[0])
        fire(c0 + 2, slots[0])
        finish(c0 + 1, slots[1])
        return carry

    lax.fori_loop(0, NPAIR, pair, 0)
    finish(NCH - 1, slots[0])


def kernel(z, edge_index, edge_type, rel_emb):
    src = edge_index[0].astype(jnp.int32)
    dst = edge_index[1].astype(jnp.int32)
    et = edge_type.astype(jnp.int32)
    return _transe(z, src, dst, et, rel_emb)


# speedup vs baseline: 5.5115x; 4.5878x over previous
import functools

import jax
import jax.numpy as jnp
from jax import lax
from jax.experimental import pallas as pl
from jax.experimental.pallas import tpu as pltpu
from jax.experimental.pallas import tpu_sc as plsc

E = 320000
H = 128
NW = 32
EPW = E // NW
C = 80
NCH = EPW // C
NPAIR = (NCH - 1) // 2
G = C // 16

_mesh = plsc.VectorSubcoreMesh(core_axis_name="c", subcore_axis_name="s")

_slot_types = [
    pltpu.VMEM((C,), jnp.int32),
    pltpu.VMEM((C,), jnp.int32),
    pltpu.VMEM((C,), jnp.int32),
    pltpu.VMEM((C, H), jnp.float32),
    pltpu.VMEM((C, H), jnp.float32),
    pltpu.VMEM((C, H), jnp.float32),
    pltpu.VMEM((C,), jnp.float32),
    pltpu.SemaphoreType.DMA,
]


@functools.partial(
    pl.kernel,
    out_type=jax.ShapeDtypeStruct((E,), jnp.float32),
    mesh=_mesh,
    compiler_params=pltpu.CompilerParams(needs_layout_passes=False),
    scratch_types=_slot_types + _slot_types,
)
def _transe(z_h, src_h, dst_h, et_h, rel_h, out_h, *scratch):
    slots = (scratch[:8], scratch[8:])
    wid = lax.axis_index("s") * 2 + lax.axis_index("c")
    base = wid * EPW

    def fire(ci, s):
        si, di, ti, a, b, r, _, sem = s
        off = base + ci * C
        pltpu.sync_copy(src_h.at[pl.ds(off, C)], si)
        pltpu.sync_copy(dst_h.at[pl.ds(off, C)], di)
        pltpu.sync_copy(et_h.at[pl.ds(off, C)], ti)
        pltpu.make_async_copy(z_h.at[si], a, sem).start()
        pltpu.make_async_copy(z_h.at[di], b, sem).start()
        pltpu.make_async_copy(rel_h.at[ti], r, sem).start()

    def finish(ci, s):
        si, di, ti, a, b, r, o, sem = s
        pltpu.make_async_copy(z_h.at[si], a, sem).wait()
        pltpu.make_async_copy(z_h.at[di], b, sem).wait()
        pltpu.make_async_copy(rel_h.at[ti], r, sem).wait()

        def group(g, carry):
            lane = lax.iota(jnp.int32, 16)
            rows = g * 16 + lane
            acc = jnp.zeros((16,), jnp.float32)
            for f in range(H):
                fv = (lane + f) & (H - 1)
                va = plsc.load_gather(a, [rows, fv])
                vb = plsc.load_gather(b, [rows, fv])
                vr = plsc.load_gather(r, [rows, fv])
                d = va + vr - vb
                acc = acc + d * d
            ibits = plsc.bitcast(acc, jnp.int32)
            magic = jnp.full((16,), 0x5F3759DF, jnp.int32)
            y = plsc.bitcast(magic - (ibits >> 1), jnp.float32)
            for _ in range(3):
                y = y * (1.5 - 0.5 * acc * y * y)
            res = jnp.where(acc > 0.0, -(acc * y), 0.0)
            o[pl.ds(g * 16, 16)] = res
            return carry

        lax.fori_loop(0, G, group, 0)
        pltpu.sync_copy(o, out_h.at[pl.ds(base + ci * C, C)])

    fire(0, slots[0])

    def pair(j, carry):
        c0 = j * 2
        fire(c0 + 1, slots[1])
        finish(c0, slots[0])
        fire(c0 + 2, slots[0])
        finish(c0 + 1, slots[1])
        return carry

    lax.fori_loop(0, NPAIR, pair, 0)
    finish(NCH - 1, slots[0])


def kernel(z, edge_index, edge_type, rel_emb):
    src = edge_index[0].astype(jnp.int32)
    dst = edge_index[1].astype(jnp.int32)
    et = edge_type.astype(jnp.int32)
    return _transe(z, src, dst, et, rel_emb)


# speedup vs baseline: 9.0768x; 1.6469x over previous
import functools

import jax
import jax.numpy as jnp
from jax import lax
from jax.experimental import pallas as pl
from jax.experimental.pallas import tpu as pltpu
from jax.experimental.pallas import tpu_sc as plsc

E = 320000
H = 128
NW = 32
EPW = E // NW
C = 80
NCH = EPW // C
NTRI = (NCH - 2) // 3
G = C // 16

_mesh = plsc.VectorSubcoreMesh(core_axis_name="c", subcore_axis_name="s")

_slot_types = [
    pltpu.VMEM((C, H), jnp.float32),
    pltpu.VMEM((C, H), jnp.float32),
    pltpu.SemaphoreType.DMA,
    pltpu.SemaphoreType.DMA,
]


@functools.partial(
    pl.kernel,
    out_type=jax.ShapeDtypeStruct((E,), jnp.float32),
    mesh=_mesh,
    compiler_params=pltpu.CompilerParams(needs_layout_passes=False),
    scratch_types=[
        pltpu.VMEM((EPW,), jnp.int32),
        pltpu.VMEM((EPW,), jnp.int32),
        pltpu.VMEM((EPW,), jnp.int32),
        pltpu.VMEM((EPW,), jnp.float32),
    ] + _slot_types + _slot_types + _slot_types,
)
def _transe(z_h, src_h, dst_h, et_h, rel_h, out_h, si, di, ti, o, *scratch):
    slots = (scratch[0:4], scratch[4:8], scratch[8:12])
    wid = lax.axis_index("s") * 2 + lax.axis_index("c")
    base = wid * EPW

    pltpu.sync_copy(src_h.at[pl.ds(base, EPW)], si)
    pltpu.sync_copy(dst_h.at[pl.ds(base, EPW)], di)
    pltpu.sync_copy(et_h.at[pl.ds(base, EPW)], ti)

    def fire1(ci, s):
        a, b, sem_g, _ = s
        off = ci * C
        pltpu.make_async_copy(z_h.at[si.at[pl.ds(off, C)]], a, sem_g).start()
        pltpu.make_async_copy(z_h.at[di.at[pl.ds(off, C)]], b, sem_g).start()

    def fire2(ci, s):
        a, b, sem_g, sem_a = s
        pltpu.make_async_copy(z_h.at[si.at[pl.ds(ci * C, C)]], a, sem_g).wait()
        pltpu.make_async_copy(z_h.at[di.at[pl.ds(ci * C, C)]], b, sem_g).wait()
        pltpu.async_copy(rel_h.at[ti.at[pl.ds(ci * C, C)]], a, sem_a, add=True)

    def finish(ci, s):
        a, b, _, sem_a = s
        pltpu.make_async_copy(rel_h.at[ti.at[pl.ds(ci * C, C)]], a, sem_a).wait()

        def group(g, carry):
            lane = lax.iota(jnp.int32, 16)
            rows = g * 16 + lane
            FB = 32

            def fblock(fb, acc):
                for fo in range(FB):
                    fv = (lane + (fb * FB + fo)) & (H - 1)
                    va = plsc.load_gather(a, [rows, fv])
                    vb = plsc.load_gather(b, [rows, fv])
                    d = va - vb
                    acc = acc + d * d
                return acc

            acc = lax.fori_loop(0, H // FB, fblock, jnp.zeros((16,), jnp.float32))
            ibits = plsc.bitcast(acc, jnp.int32)
            magic = jnp.full((16,), 0x5F3759DF, jnp.int32)
            y = plsc.bitcast(magic - (ibits >> 1), jnp.float32)
            for _ in range(3):
                y = y * (1.5 - 0.5 * acc * y * y)
            res = jnp.where(acc > 0.0, -(acc * y), 0.0)
            o[pl.ds(ci * C + g * 16, 16)] = res
            return carry

        lax.fori_loop(0, G, group, 0)

    fire1(0, slots[0])
    fire2(0, slots[0])
    fire1(1, slots[1])

    def triple(j, carry):
        c0 = j * 3
        for k in range(3):
            ci = c0 + k
            fire1(ci + 2, slots[(k + 2) % 3])
            fire2(ci + 1, slots[(k + 1) % 3])
            finish(ci, slots[k])
        return carry

    lax.fori_loop(0, NTRI, triple, 0)
    fire2(NCH - 1, slots[(NCH - 1) % 3])
    finish(NCH - 2, slots[(NCH - 2) % 3])
    finish(NCH - 1, slots[(NCH - 1) % 3])

    pltpu.sync_copy(o, out_h.at[pl.ds(base, EPW)])


def kernel(z, edge_index, edge_type, rel_emb):
    src = edge_index[0].astype(jnp.int32)
    dst = edge_index[1].astype(jnp.int32)
    et = edge_type.astype(jnp.int32)
    return _transe(z, src, dst, et, rel_emb)


# speedup vs baseline: 10.4475x; 1.1510x over previous
import functools

import jax
import jax.numpy as jnp
from jax import lax
from jax.experimental import pallas as pl
from jax.experimental.pallas import tpu as pltpu
from jax.experimental.pallas import tpu_sc as plsc

E = 320000
H = 128
NW = 32
EPW = E // NW
C = 80
NCH = EPW // C
NTRI = (NCH - 2) // 3
G = C // 16

_mesh = plsc.VectorSubcoreMesh(core_axis_name="c", subcore_axis_name="s")

_slot_types = [
    pltpu.VMEM((C, H), jnp.float32),
    pltpu.VMEM((C, H), jnp.float32),
    pltpu.SemaphoreType.DMA,
    pltpu.SemaphoreType.DMA,
]


@functools.partial(
    pl.kernel,
    out_type=jax.ShapeDtypeStruct((E,), jnp.float32),
    mesh=_mesh,
    compiler_params=pltpu.CompilerParams(needs_layout_passes=False),
    scratch_types=[
        pltpu.VMEM((EPW,), jnp.int32),
        pltpu.VMEM((EPW,), jnp.int32),
        pltpu.VMEM((EPW,), jnp.int32),
        pltpu.VMEM((EPW,), jnp.float32),
        pltpu.VMEM_SHARED((500, H), jnp.float32),
    ] + _slot_types + _slot_types + _slot_types,
)
def _transe(z_h, src_h, dst_h, et_h, rel_h, out_h, si, di, ti, o, rel_sp, *scratch):
    slots = (scratch[0:4], scratch[4:8], scratch[8:12])
    wid = lax.axis_index("s") * 2 + lax.axis_index("c")
    base = wid * EPW

    @pl.when(lax.axis_index("s") == 0)
    def _():
        pltpu.sync_copy(rel_h, rel_sp)

    pltpu.sync_copy(src_h.at[pl.ds(base, EPW)], si)
    pltpu.sync_copy(dst_h.at[pl.ds(base, EPW)], di)
    pltpu.sync_copy(et_h.at[pl.ds(base, EPW)], ti)
    plsc.subcore_barrier()

    def fire1(ci, s):
        a, b, sem_g, _ = s
        off = ci * C
        pltpu.make_async_copy(z_h.at[si.at[pl.ds(off, C)]], a, sem_g).start()
        pltpu.make_async_copy(z_h.at[di.at[pl.ds(off, C)]], b, sem_g).start()

    def fire2(ci, s):
        a, b, sem_g, sem_a = s
        pltpu.make_async_copy(z_h.at[si.at[pl.ds(ci * C, C)]], a, sem_g).wait()
        pltpu.make_async_copy(z_h.at[di.at[pl.ds(ci * C, C)]], b, sem_g).wait()
        pltpu.async_copy(rel_sp.at[ti.at[pl.ds(ci * C, C)]], a, sem_a, add=True)

    def finish(ci, s):
        a, b, _, sem_a = s
        pltpu.make_async_copy(rel_sp.at[ti.at[pl.ds(ci * C, C)]], a, sem_a).wait()

        def group(g, carry):
            lane = lax.iota(jnp.int32, 16)
            rows = g * 16 + lane
            FB = 32

            def fblock(fb, acc):
                for fo in range(FB):
                    fv = (lane + (fb * FB + fo)) & (H - 1)
                    va = plsc.load_gather(a, [rows, fv])
                    vb = plsc.load_gather(b, [rows, fv])
                    d = va - vb
                    acc = acc + d * d
                return acc

            acc = lax.fori_loop(0, H // FB, fblock, jnp.zeros((16,), jnp.float32))
            ibits = plsc.bitcast(acc, jnp.int32)
            magic = jnp.full((16,), 0x5F3759DF, jnp.int32)
            y = plsc.bitcast(magic - (ibits >> 1), jnp.float32)
            for _ in range(3):
                y = y * (1.5 - 0.5 * acc * y * y)
            res = jnp.where(acc > 0.0, -(acc * y), 0.0)
            o[pl.ds(ci * C + g * 16, 16)] = res
            return carry

        lax.fori_loop(0, G, group, 0)

    fire1(0, slots[0])
    fire2(0, slots[0])
    fire1(1, slots[1])

    def triple(j, carry):
        c0 = j * 3
        for k in range(3):
            ci = c0 + k
            fire1(ci + 2, slots[(k + 2) % 3])
            fire2(ci + 1, slots[(k + 1) % 3])
            finish(ci, slots[k])
        return carry

    lax.fori_loop(0, NTRI, triple, 0)
    fire2(NCH - 1, slots[(NCH - 1) % 3])
    finish(NCH - 2, slots[(NCH - 2) % 3])
    finish(NCH - 1, slots[(NCH - 1) % 3])

    pltpu.sync_copy(o, out_h.at[pl.ds(base, EPW)])


def kernel(z, edge_index, edge_type, rel_emb):
    src = edge_index[0].astype(jnp.int32)
    dst = edge_index[1].astype(jnp.int32)
    et = edge_type.astype(jnp.int32)
    return _transe(z, src, dst, et, rel_emb)


# speedup vs baseline: 11.5319x; 1.1038x over previous
import functools

import jax
import jax.numpy as jnp
from jax import lax
from jax.experimental import pallas as pl
from jax.experimental.pallas import tpu as pltpu
from jax.experimental.pallas import tpu_sc as plsc

E = 320000
H = 128
NW = 32
EPW = E // NW
C = 80
NCH = EPW // C
NQUAD = (NCH - 5) // 4
G = C // 16

_mesh = plsc.VectorSubcoreMesh(core_axis_name="c", subcore_axis_name="s")

_slot_types = [
    pltpu.VMEM((C, H), jnp.float32),
    pltpu.SemaphoreType.DMA,
]


@functools.partial(
    pl.kernel,
    out_type=jax.ShapeDtypeStruct((E,), jnp.float32),
    mesh=_mesh,
    compiler_params=pltpu.CompilerParams(needs_layout_passes=False),
    scratch_types=[
        pltpu.VMEM((EPW,), jnp.int32),
        pltpu.VMEM((EPW,), jnp.int32),
        pltpu.VMEM((EPW,), jnp.int32),
        pltpu.VMEM((EPW,), jnp.float32),
        pltpu.VMEM_SHARED((500, H), jnp.float32),
    ] + _slot_types + _slot_types + _slot_types + _slot_types,
)
def _transe(z_h, zn_h, src_h, dst_h, et_h, rel_h, out_h, si, di, ti, o, rel_sp,
            *scratch):
    slots = tuple(scratch[2 * k:2 * k + 2] for k in range(4))
    sid = lax.axis_index("s")
    wid = sid * 2 + lax.axis_index("c")
    base = wid * EPW

    @pl.when(sid == 0)
    def _():
        pltpu.sync_copy(rel_h, rel_sp)

    pltpu.sync_copy(src_h.at[pl.ds(base, EPW)], si)
    pltpu.sync_copy(dst_h.at[pl.ds(base, EPW)], di)
    pltpu.sync_copy(et_h.at[pl.ds(base, EPW)], ti)
    plsc.subcore_barrier()

    def fire1(ci, s):
        d, sem = s
        pltpu.make_async_copy(z_h.at[si.at[pl.ds(ci * C, C)]], d, sem).start()

    def fire2(ci, s):
        d, sem = s
        pltpu.make_async_copy(z_h.at[si.at[pl.ds(ci * C, C)]], d, sem).wait()
        pltpu.async_copy(rel_sp.at[ti.at[pl.ds(ci * C, C)]], d, sem, add=True)

    def fire3(ci, s):
        d, sem = s
        pltpu.make_async_copy(rel_sp.at[ti.at[pl.ds(ci * C, C)]], d, sem).wait()
        pltpu.async_copy(zn_h.at[di.at[pl.ds(ci * C, C)]], d, sem, add=True)

    def finish(ci, s):
        d, sem = s
        pltpu.make_async_copy(zn_h.at[di.at[pl.ds(ci * C, C)]], d, sem).wait()

        def group(g, carry):
            lane = lax.iota(jnp.int32, 16)
            rows = g * 16 + lane
            FB = 32

            def fblock(fb, acc):
                for fo in range(FB):
                    fv = (lane + (fb * FB + fo)) & (H - 1)
                    vd = plsc.load_gather(d, [rows, fv])
                    acc = acc + vd * vd
                return acc

            acc = lax.fori_loop(0, H // FB, fblock, jnp.zeros((16,), jnp.float32))
            ibits = plsc.bitcast(acc, jnp.int32)
            magic = jnp.full((16,), 0x5F3759DF, jnp.int32)
            y = plsc.bitcast(magic - (ibits >> 1), jnp.float32)
            for _ in range(3):
                y = y * (1.5 - 0.5 * acc * y * y)
            res = jnp.where(acc > 0.0, -(acc * y), 0.0)
            o[pl.ds(ci * C + g * 16, 16)] = res
            return carry

        lax.fori_loop(0, G, group, 0)

    fire1(0, slots[0])
    fire2(0, slots[0])
    fire1(1, slots[1])
    fire3(0, slots[0])
    fire2(1, slots[1])
    fire1(2, slots[2])

    def quad(j, carry):
        c0 = j * 4
        for k in range(4):
            ci = c0 + k
            fire1(ci + 3, slots[(k + 3) % 4])
            fire2(ci + 2, slots[(k + 2) % 4])
            fire3(ci + 1, slots[(k + 1) % 4])
            finish(ci, slots[k])
        return carry

    lax.fori_loop(0, NQUAD, quad, 0)
    for ci in range(NQUAD * 4, NCH):
        if ci + 3 < NCH:
            fire1(ci + 3, slots[(ci + 3) % 4])
        if ci + 2 < NCH:
            fire2(ci + 2, slots[(ci + 2) % 4])
        if ci + 1 < NCH:
            fire3(ci + 1, slots[(ci + 1) % 4])
        finish(ci, slots[ci % 4])

    pltpu.sync_copy(o, out_h.at[pl.ds(base, EPW)])


def kernel(z, edge_index, edge_type, rel_emb):
    src = edge_index[0].astype(jnp.int32)
    dst = edge_index[1].astype(jnp.int32)
    et = edge_type.astype(jnp.int32)
    return _transe(z, -z, src, dst, et, rel_emb)
